# baseline (speedup 1.0000x reference)
import jax
import jax.numpy as jnp
from jax import lax
from jax.experimental import pallas as pl
from jax.experimental.pallas import tpu as pltpu
from jax.experimental.pallas import tpu_sc as plsc

B = 32
P = 8732
C = 21
P2 = 8832
PADN = P2 - P
CPAD = 24
NEGPOS_RATIO = 3
THRESHHOLD = 0.1
NGT = 16

_SC_PARAMS = pltpu.CompilerParams(needs_layout_passes=False)


def _match_body(priors_ref, truth_ref, ploc_ref,
                conf_ref, npos_ref, lloc_ref):
    b = pl.program_id(0)
    lane = lax.broadcasted_iota(jnp.int32, (1, P), 1).astype(jnp.float32)

    pr = priors_ref[...]
    pcx, pcy, pw, ph = pr[0:1], pr[1:2], pr[2:3], pr[3:4]
    px1 = pcx - 0.5 * pw
    py1 = pcy - 0.5 * ph
    px2 = pcx + 0.5 * pw
    py2 = pcy + 0.5 * ph
    size_p = (px2 - px1) * (py2 - py1)

    t = truth_ref[0]
    tx1, ty1 = t[:, 0:1], t[:, 1:2]
    tx2, ty2 = t[:, 2:3], t[:, 3:4]
    tlab = t[:, 4:5]
    size_g = (tx2 - tx1) * (ty2 - ty1)

    ix1 = jnp.maximum(px1, tx1)
    iy1 = jnp.maximum(py1, ty1)
    ix2 = jnp.minimum(px2, tx2)
    iy2 = jnp.minimum(py2, ty2)
    cw = jnp.maximum(ix2 - ix1, 0.0)
    chh = jnp.maximum(iy2 - iy1, 0.0)
    cross = cw * chh
    total = size_p + size_g
    iou = cross / total - cross

    giota = lax.broadcasted_iota(jnp.int32, (NGT, 1), 0).astype(jnp.float32)

    best_gt_overlap = jnp.max(iou, axis=0, keepdims=True)
    eq_g = iou == best_gt_overlap
    bgi = jnp.min(jnp.where(eq_g, giota, 99.0), axis=0, keepdims=True)

    best_p_overlap = jnp.max(iou, axis=1, keepdims=True)
    eq_p = iou == best_p_overlap
    bpi = jnp.min(jnp.where(eq_p, lane, 1e9), axis=1, keepdims=True)

    hit = lane == bpi
    winner = jnp.max(jnp.where(hit, giota, -1.0), axis=0, keepdims=True)
    bgi = jnp.where(winner >= 0.0, winner, bgi)

    onehot = (bgi == giota).astype(jnp.float32)
    gath = jax.lax.dot_general(
        t, onehot, (((0,), (0,)), ((), ())),
        preferred_element_type=jnp.float32)
    mx1, my1 = gath[0:1], gath[1:2]
    mx2, my2 = gath[2:3], gath[3:4]
    conf = gath[4:5] + 1.0

    conf = jnp.where(best_gt_overlap < THRESHHOLD, 0.0, conf)
    pos = conf > 0.0

    l_cx = ((mx1 + mx2) / 2.0 - pcx) / pw
    l_cy = ((my1 + my2) / 2.0 - pcy) / ph
    l_w = jnp.log((mx2 - mx1) / pw)
    l_h = jnp.log((my2 - my1) / ph)

    pd = ploc_ref[0]
    d0 = jnp.abs(pd[0:1] - l_cx)
    d1 = jnp.abs(pd[1:2] - l_cy)
    d2 = jnp.abs(pd[2:3] - l_w)
    d3 = jnp.abs(pd[3:4] - l_h)

    def smooth(dd):
        return jnp.where(dd < 1.0, 0.5 * dd * dd, dd - 0.5)

    ssum = smooth(d0) + smooth(d1) + smooth(d2) + smooth(d3)
    lloc_part = jnp.sum(jnp.where(pos, ssum, 0.0))
    npos_part = jnp.sum(jnp.where(pos, 1.0, 0.0))

    conf_ref[0] = jnp.concatenate(
        [conf, jnp.zeros((1, PADN), jnp.float32)], axis=1)
    npos_ref[0, 0, 0] = npos_part

    @pl.when(b == 0)
    def _():
        lloc_ref[0, 0] = 0.0

    lloc_ref[0, 0] += lloc_part


def _run_match(priors_t, truth, ploc_t):
    return pl.pallas_call(
        _match_body,
        grid=(B,),
        in_specs=[
            pl.BlockSpec((4, P), lambda b: (0, 0)),
            pl.BlockSpec((1, NGT, 5), lambda b: (b, 0, 0)),
            pl.BlockSpec((1, 4, P), lambda b: (b, 0, 0)),
        ],
        out_specs=[
            pl.BlockSpec((1, 1, P2), lambda b: (b, 0, 0)),
            pl.BlockSpec((1, 1, 1), lambda b: (b, 0, 0),
                         memory_space=pltpu.SMEM),
            pl.BlockSpec((1, 1), lambda b: (0, 0),
                         memory_space=pltpu.SMEM),
        ],
        out_shape=[
            jax.ShapeDtypeStruct((B, 1, P2), jnp.float32),
            jax.ShapeDtypeStruct((B, 1, 1), jnp.float32),
            jax.ShapeDtypeStruct((1, 1), jnp.float32),
        ],
        compiler_params=pltpu.CompilerParams(
            dimension_semantics=("arbitrary",)),
    )(priors_t, truth, ploc_t)


def _keys_body(pconf_ref, conf_ref,
               keys_ref, ce2_ref, cepos_ref, gmax_ref):
    j = pl.program_id(0)
    b = pl.program_id(1)
    x = pconf_ref[0]

    @pl.when((j == 0) & (b == 0))
    def _():
        gmax_ref[0] = -jnp.inf

    @pl.when(j == 0)
    def _():
        gmax_ref[0] = jnp.maximum(gmax_ref[0], jnp.max(x))

    @pl.when(j == 1)
    def _keys_phase():
        _keys_phase1(x, conf_ref, keys_ref, ce2_ref, cepos_ref, gmax_ref,
                     pos_b=b)


def _keys_phase1(x, conf_ref, keys_ref, ce2_ref, cepos_ref, gmax_ref,
                 pos_b):
    xmax = gmax_ref[0]
    e2 = jnp.exp(x - xmax)
    s2 = jnp.sum(e2, axis=0, keepdims=True)
    lse = jnp.log(s2) + xmax

    conf = conf_ref[0][:, :P]
    pos = conf > 0.0
    ciota = lax.broadcasted_iota(jnp.int32, (C, 1), 0).astype(jnp.float32)
    sel = ciota == conf
    picked = jnp.sum(jnp.where(sel, x, 0.0), axis=0, keepdims=True)

    ce = lse - picked
    key = jnp.where(pos, 0.0, ce)
    cepos_part = jnp.sum(jnp.where(pos, ce, 0.0))

    keys_ref[0] = jnp.concatenate(
        [key, jnp.full((1, PADN), jnp.inf, jnp.float32)], axis=1)
    ce2_ref[0] = jnp.concatenate(
        [key, jnp.zeros((1, PADN), jnp.float32)], axis=1)

    @pl.when(pos_b == 0)
    def _():
        cepos_ref[0, 0] = 0.0

    cepos_ref[0, 0] += cepos_part


def _run_keys(pconf_t, conf):
    return pl.pallas_call(
        _keys_body,
        grid=(2, B),
        in_specs=[
            pl.BlockSpec((1, C, P), lambda j, b: (b, 0, 0)),
            pl.BlockSpec((1, 1, P2), lambda j, b: (b, 0, 0)),
        ],
        out_specs=[
            pl.BlockSpec((1, 1, P2), lambda j, b: (b, 0, 0)),
            pl.BlockSpec((1, 1, P2), lambda j, b: (b, 0, 0)),
            pl.BlockSpec((1, 1), lambda j, b: (0, 0),
                         memory_space=pltpu.SMEM),
        ],
        out_shape=[
            jax.ShapeDtypeStruct((B, 1, P2), jnp.float32),
            jax.ShapeDtypeStruct((B, 1, P2), jnp.float32),
            jax.ShapeDtypeStruct((1, 1), jnp.float32),
        ],
        scratch_shapes=[pltpu.SMEM((1,), jnp.float32)],
        compiler_params=pltpu.CompilerParams(
            dimension_semantics=("arbitrary", "arbitrary")),
    )(pconf_t, conf)


_NV = P2 // 16
_NBUCKET = 2048
_NHV = _NBUCKET // 16


def _sc_body(keys_hbm, ce2_hbm, npos_hbm, t_hbm,
             kf, k0, i0, k1, i1, hist, ce2s, accv, nposv):
    wid = lax.axis_index("s") * 2 + lax.axis_index("c")
    pltpu.sync_copy(keys_hbm.at[wid, 0], kf)
    pltpu.sync_copy(ce2_hbm.at[wid, 0], ce2s)
    pltpu.sync_copy(npos_hbm, nposv)

    lane16 = lax.iota(jnp.int32, 16)
    sign = jnp.full((16,), jnp.int32(-2147483648))

    def init_body(v, _):
        kb = plsc.bitcast(kf[pl.ds(v * 16, 16)], jnp.int32)
        m = lax.shift_right_arithmetic(kb, 31)
        u = lax.bitwise_xor(kb, lax.bitwise_or(m, sign))
        k0[pl.ds(v * 16, 16)] = u
        i0[pl.ds(v * 16, 16)] = v * 16 + lane16
        return 0

    lax.fori_loop(0, _NV, init_body, 0)

    npf = plsc.load_gather(nposv, [jnp.full((16,), wid, jnp.int32)])
    nneg = jnp.minimum(
        jnp.float32(NEGPOS_RATIO) * npf,
        jnp.float32(P - 1)).astype(jnp.int32)

    bufs = [(k0, i0), (k1, i1), (k0, i0), (k1, i1)]
    for p, shift in enumerate((0, 11, 22)):
        src_k, src_i = bufs[p]
        dst_k, dst_i = bufs[p + 1]
        is_last = p == 2

        def zero_body(h, _):
            hist[pl.ds(h * 16, 16)] = jnp.zeros((16,), jnp.int32)
            return 0

        lax.fori_loop(0, _NHV, zero_body, 0)

        def hist_body(v, _, src_k=src_k, shift=shift):
            k = src_k[pl.ds(v * 16, 16)]
            d = lax.bitwise_and(
                lax.shift_right_logical(k, shift), jnp.int32(_NBUCKET - 1))
            cnt, last = plsc.scan_count(d)
            plsc.addupdate_scatter(hist, [d], cnt, mask=last)
            return 0

        lax.fori_loop(0, _NV, hist_body, 0)

        def scan_body(h, run):
            v = hist[pl.ds(h * 16, 16)]
            cs = plsc.cumsum(v)
            hist[pl.ds(h * 16, 16)] = cs - v + run
            return run + jnp.sum(v)

        lax.fori_loop(0, _NHV, scan_body, jnp.int32(0))

        if not is_last:
            def perm_body(v, _, src_k=src_k, src_i=src_i,
                          dst_k=dst_k, dst_i=dst_i, shift=shift):
                k = src_k[pl.ds(v * 16, 16)]
                iv = src_i[pl.ds(v * 16, 16)]
                d = lax.bitwise_and(
                    lax.shift_right_logical(k, shift),
                    jnp.int32(_NBUCKET - 1))
                cnt, last = plsc.scan_count(d)
                base = plsc.load_gather(hist, [d])
                dest = base + cnt - 1
                plsc.store_scatter(dst_k, [dest], k)
                plsc.store_scatter(dst_i, [dest], iv)
                plsc.addupdate_scatter(hist, [d], cnt, mask=last)
                return 0

            lax.fori_loop(0, _NV, perm_body, 0)
        else:
            def last_body(v, acc, src_k=src_k, src_i=src_i, shift=shift):
                k = src_k[pl.ds(v * 16, 16)]
                iv = src_i[pl.ds(v * 16, 16)]
                d = lax.bitwise_and(
                    lax.shift_right_logical(k, shift),
                    jnp.int32(_NBUCKET - 1))
                cnt, last = plsc.scan_count(d)
                base = plsc.load_gather(hist, [d])
                dest = base + cnt - 1
                cv = plsc.load_gather(ce2s, [dest])
                plsc.addupdate_scatter(hist, [d], cnt, mask=last)
                return acc + jnp.where(iv < nneg, cv, 0.0)

            acc = lax.fori_loop(0, _NV, last_body,
                                jnp.zeros((16,), jnp.float32))
            accv[...] = acc
            pltpu.sync_copy(accv, t_hbm.at[wid])


def _run_sc_sort_select(keys, ce2, npos):
    mesh = plsc.VectorSubcoreMesh(core_axis_name="c", subcore_axis_name="s")
    f = pl.kernel(
        _sc_body,
        out_type=jax.ShapeDtypeStruct((B, 16), jnp.float32),
        mesh=mesh,
        scratch_types=[
            pltpu.VMEM((P2,), jnp.float32),
            pltpu.VMEM((P2,), jnp.int32),
            pltpu.VMEM((P2,), jnp.int32),
            pltpu.VMEM((P2,), jnp.int32),
            pltpu.VMEM((P2,), jnp.int32),
            pltpu.VMEM((_NBUCKET,), jnp.int32),
            pltpu.VMEM((P2,), jnp.float32),
            pltpu.VMEM((16,), jnp.float32),
            pltpu.VMEM((B,), jnp.float32),
        ],
        compiler_params=_SC_PARAMS,
    )
    return f(keys, ce2, npos)


def kernel(pred_conf, pred_loc, priory_boxes, truth):
    pconf_t = jnp.transpose(pred_conf, (0, 2, 1))
    ploc_t = jnp.transpose(pred_loc, (0, 2, 1))
    priors_t = priory_boxes.T

    conf, npos, lloc = _run_match(priors_t, truth, ploc_t)
    keys, ce2, cepos = _run_keys(pconf_t, conf)
    tpart = _run_sc_sort_select(keys, ce2, jnp.reshape(npos, (B,)))

    n = jnp.sum(npos)
    loss_loc = lloc[0, 0] / n
    loss_c = (cepos[0, 0] + jnp.sum(tpart)) / n
    return (loss_loc, loss_c)


# speedup vs baseline: 1.1121x; 1.1121x over previous
import jax
import jax.numpy as jnp
from jax import lax
from jax.experimental import pallas as pl
from jax.experimental.pallas import tpu as pltpu
from jax.experimental.pallas import tpu_sc as plsc

B = 32
P = 8732
C = 21
P2 = 8832
PADN = P2 - P
CPAD = 24
NEGPOS_RATIO = 3
THRESHHOLD = 0.1
NGT = 16

_SC_PARAMS = pltpu.CompilerParams(needs_layout_passes=False)


def _match_body(priors_ref, truth_ref, ploc_ref,
                conf_ref, npos_ref, lloc_ref):
    b = pl.program_id(0)
    lane = lax.broadcasted_iota(jnp.int32, (1, P), 1).astype(jnp.float32)

    pr = priors_ref[...]
    pcx, pcy, pw, ph = pr[0:1], pr[1:2], pr[2:3], pr[3:4]
    px1 = pcx - 0.5 * pw
    py1 = pcy - 0.5 * ph
    px2 = pcx + 0.5 * pw
    py2 = pcy + 0.5 * ph
    size_p = (px2 - px1) * (py2 - py1)

    t = truth_ref[0]
    tx1, ty1 = t[:, 0:1], t[:, 1:2]
    tx2, ty2 = t[:, 2:3], t[:, 3:4]
    tlab = t[:, 4:5]
    size_g = (tx2 - tx1) * (ty2 - ty1)

    ix1 = jnp.maximum(px1, tx1)
    iy1 = jnp.maximum(py1, ty1)
    ix2 = jnp.minimum(px2, tx2)
    iy2 = jnp.minimum(py2, ty2)
    cw = jnp.maximum(ix2 - ix1, 0.0)
    chh = jnp.maximum(iy2 - iy1, 0.0)
    cross = cw * chh
    total = size_p + size_g
    iou = cross / total - cross

    giota = lax.broadcasted_iota(jnp.int32, (NGT, 1), 0).astype(jnp.float32)

    best_gt_overlap = jnp.max(iou, axis=0, keepdims=True)
    eq_g = iou == best_gt_overlap
    bgi = jnp.min(jnp.where(eq_g, giota, 99.0), axis=0, keepdims=True)

    best_p_overlap = jnp.max(iou, axis=1, keepdims=True)
    eq_p = iou == best_p_overlap
    bpi = jnp.min(jnp.where(eq_p, lane, 1e9), axis=1, keepdims=True)

    hit = lane == bpi
    winner = jnp.max(jnp.where(hit, giota, -1.0), axis=0, keepdims=True)
    bgi = jnp.where(winner >= 0.0, winner, bgi)

    onehot = (bgi == giota).astype(jnp.float32)
    gath = jax.lax.dot_general(
        t, onehot, (((0,), (0,)), ((), ())),
        preferred_element_type=jnp.float32)
    mx1, my1 = gath[0:1], gath[1:2]
    mx2, my2 = gath[2:3], gath[3:4]
    conf = gath[4:5] + 1.0

    conf = jnp.where(best_gt_overlap < THRESHHOLD, 0.0, conf)
    pos = conf > 0.0

    l_cx = ((mx1 + mx2) / 2.0 - pcx) / pw
    l_cy = ((my1 + my2) / 2.0 - pcy) / ph
    l_w = jnp.log((mx2 - mx1) / pw)
    l_h = jnp.log((my2 - my1) / ph)

    pd = ploc_ref[0]
    d0 = jnp.abs(pd[0:1] - l_cx)
    d1 = jnp.abs(pd[1:2] - l_cy)
    d2 = jnp.abs(pd[2:3] - l_w)
    d3 = jnp.abs(pd[3:4] - l_h)

    def smooth(dd):
        return jnp.where(dd < 1.0, 0.5 * dd * dd, dd - 0.5)

    ssum = smooth(d0) + smooth(d1) + smooth(d2) + smooth(d3)
    lloc_part = jnp.sum(jnp.where(pos, ssum, 0.0))
    npos_part = jnp.sum(jnp.where(pos, 1.0, 0.0))

    conf_ref[0] = jnp.concatenate(
        [conf, jnp.zeros((1, PADN), jnp.float32)], axis=1)
    npos_ref[0, 0, 0] = npos_part

    @pl.when(b == 0)
    def _():
        lloc_ref[0, 0] = 0.0

    lloc_ref[0, 0] += lloc_part


def _run_match(priors_t, truth, ploc_t):
    return pl.pallas_call(
        _match_body,
        grid=(B,),
        in_specs=[
            pl.BlockSpec((4, P), lambda b: (0, 0)),
            pl.BlockSpec((1, NGT, 5), lambda b: (b, 0, 0)),
            pl.BlockSpec((1, 4, P), lambda b: (b, 0, 0)),
        ],
        out_specs=[
            pl.BlockSpec((1, 1, P2), lambda b: (b, 0, 0)),
            pl.BlockSpec((1, 1, 1), lambda b: (b, 0, 0),
                         memory_space=pltpu.SMEM),
            pl.BlockSpec((1, 1), lambda b: (0, 0),
                         memory_space=pltpu.SMEM),
        ],
        out_shape=[
            jax.ShapeDtypeStruct((B, 1, P2), jnp.float32),
            jax.ShapeDtypeStruct((B, 1, 1), jnp.float32),
            jax.ShapeDtypeStruct((1, 1), jnp.float32),
        ],
        compiler_params=pltpu.CompilerParams(
            dimension_semantics=("arbitrary",)),
    )(priors_t, truth, ploc_t)


def _keys_body(pconf_ref, conf_ref,
               keys_ref, ce2_ref, cepos_ref, gmax_ref):
    j = pl.program_id(0)
    b = pl.program_id(1)
    x = pconf_ref[0]

    @pl.when((j == 0) & (b == 0))
    def _():
        gmax_ref[0] = -jnp.inf

    @pl.when(j == 0)
    def _():
        gmax_ref[0] = jnp.maximum(gmax_ref[0], jnp.max(x))

    @pl.when(j == 1)
    def _keys_phase():
        _keys_phase1(x, conf_ref, keys_ref, ce2_ref, cepos_ref, gmax_ref,
                     pos_b=b)


def _keys_phase1(x, conf_ref, keys_ref, ce2_ref, cepos_ref, gmax_ref,
                 pos_b):
    xmax = gmax_ref[0]
    e2 = jnp.exp(x - xmax)
    s2 = jnp.sum(e2, axis=0, keepdims=True)
    lse = jnp.log(s2) + xmax

    conf = conf_ref[0][:, :P]
    pos = conf > 0.0
    ciota = lax.broadcasted_iota(jnp.int32, (C, 1), 0).astype(jnp.float32)
    sel = ciota == conf
    picked = jnp.sum(jnp.where(sel, x, 0.0), axis=0, keepdims=True)

    ce = lse - picked
    key = jnp.where(pos, 0.0, ce)
    cepos_part = jnp.sum(jnp.where(pos, ce, 0.0))

    keys_ref[0] = jnp.concatenate(
        [key, jnp.full((1, PADN), jnp.inf, jnp.float32)], axis=1)
    ce2_ref[0] = jnp.concatenate(
        [key, jnp.zeros((1, PADN), jnp.float32)], axis=1)

    @pl.when(pos_b == 0)
    def _():
        cepos_ref[0, 0] = 0.0

    cepos_ref[0, 0] += cepos_part


def _run_keys(pconf_t, conf):
    return pl.pallas_call(
        _keys_body,
        grid=(2, B),
        in_specs=[
            pl.BlockSpec((1, C, P), lambda j, b: (b, 0, 0)),
            pl.BlockSpec((1, 1, P2), lambda j, b: (b, 0, 0)),
        ],
        out_specs=[
            pl.BlockSpec((1, 1, P2), lambda j, b: (b, 0, 0)),
            pl.BlockSpec((1, 1, P2), lambda j, b: (b, 0, 0)),
            pl.BlockSpec((1, 1), lambda j, b: (0, 0),
                         memory_space=pltpu.SMEM),
        ],
        out_shape=[
            jax.ShapeDtypeStruct((B, 1, P2), jnp.float32),
            jax.ShapeDtypeStruct((B, 1, P2), jnp.float32),
            jax.ShapeDtypeStruct((1, 1), jnp.float32),
        ],
        scratch_shapes=[pltpu.SMEM((1,), jnp.float32)],
        compiler_params=pltpu.CompilerParams(
            dimension_semantics=("arbitrary", "arbitrary")),
    )(pconf_t, conf)


_NV = P2 // 16
_NBUCKET = 2048
_NHV = _NBUCKET // 16


def _sc_body(keys_hbm, ce2_hbm, npos_hbm, t_hbm,
             kf, k0, i0, k1, i1, hista, histb, ce2s, accv, nposv):
    wid = lax.axis_index("s") * 2 + lax.axis_index("c")
    pltpu.sync_copy(keys_hbm.at[wid, 0], kf)
    pltpu.sync_copy(ce2_hbm.at[wid, 0], ce2s)
    pltpu.sync_copy(npos_hbm, nposv)

    lane16 = lax.iota(jnp.int32, 16)
    sign = jnp.full((16,), jnp.int32(-2147483648))
    npf = plsc.load_gather(nposv, [jnp.full((16,), wid, jnp.int32)])
    nneg = jnp.minimum(
        jnp.float32(NEGPOS_RATIO) * npf,
        jnp.float32(P - 1)).astype(jnp.int32)

    def digit(k, shift):
        return lax.bitwise_and(
            lax.shift_right_logical(k, shift), jnp.int32(_NBUCKET - 1))

    def zero_hist(h_ref):
        def zb(h, _):
            h_ref[pl.ds(h * 16, 16)] = jnp.zeros((16,), jnp.int32)
            return 0
        lax.fori_loop(0, _NHV, zb, 0)

    def scan_hist(h_ref):
        def sb(h, run):
            v = h_ref[pl.ds(h * 16, 16)]
            cs = plsc.cumsum(v)
            h_ref[pl.ds(h * 16, 16)] = cs - v + run
            return run + jnp.sum(v)
        lax.fori_loop(0, _NHV, sb, jnp.int32(0))

    zero_hist(hista)

    def init_body(v, _):
        kb = plsc.bitcast(kf[pl.ds(v * 16, 16)], jnp.int32)
        m = lax.shift_right_arithmetic(kb, 31)
        u = lax.bitwise_xor(kb, lax.bitwise_or(m, sign))
        k0[pl.ds(v * 16, 16)] = u
        i0[pl.ds(v * 16, 16)] = v * 16 + lane16
        d = digit(u, 0)
        cnt, last = plsc.scan_count(d)
        plsc.addupdate_scatter(hista, [d], cnt, mask=last)
        return 0

    lax.fori_loop(0, _NV, init_body, 0)

    scan_hist(hista)
    zero_hist(histb)

    def perm0_body(v, _):
        k = k0[pl.ds(v * 16, 16)]
        iv = i0[pl.ds(v * 16, 16)]
        d = digit(k, 0)
        cnt, last = plsc.scan_count(d)
        base = plsc.load_gather(hista, [d])
        dest = base + cnt - 1
        plsc.store_scatter(k1, [dest], k)
        plsc.store_scatter(i1, [dest], iv)
        plsc.addupdate_scatter(hista, [d], cnt, mask=last)
        d1 = digit(k, 11)
        cnt1, last1 = plsc.scan_count(d1)
        plsc.addupdate_scatter(histb, [d1], cnt1, mask=last1)
        return 0

    lax.fori_loop(0, _NV, perm0_body, 0)

    scan_hist(histb)
    zero_hist(hista)

    def perm1_body(v, _):
        k = k1[pl.ds(v * 16, 16)]
        iv = i1[pl.ds(v * 16, 16)]
        d = digit(k, 11)
        cnt, last = plsc.scan_count(d)
        base = plsc.load_gather(histb, [d])
        dest = base + cnt - 1
        plsc.store_scatter(k0, [dest], k)
        plsc.store_scatter(i0, [dest], iv)
        plsc.addupdate_scatter(histb, [d], cnt, mask=last)
        d2 = digit(k, 22)
        cnt2, last2 = plsc.scan_count(d2)
        plsc.addupdate_scatter(hista, [d2], cnt2, mask=last2)
        return 0

    lax.fori_loop(0, _NV, perm1_body, 0)

    scan_hist(hista)

    def last_body(v, acc):
        k = k0[pl.ds(v * 16, 16)]
        iv = i0[pl.ds(v * 16, 16)]
        d = digit(k, 22)
        cnt, last = plsc.scan_count(d)
        base = plsc.load_gather(hista, [d])
        dest = base + cnt - 1
        cv = plsc.load_gather(ce2s, [dest])
        plsc.addupdate_scatter(hista, [d], cnt, mask=last)
        return acc + jnp.where(iv < nneg, cv, 0.0)

    acc = lax.fori_loop(0, _NV, last_body, jnp.zeros((16,), jnp.float32))
    accv[...] = acc
    pltpu.sync_copy(accv, t_hbm.at[wid])


def _run_sc_sort_select(keys, ce2, npos):
    mesh = plsc.VectorSubcoreMesh(core_axis_name="c", subcore_axis_name="s")
    f = pl.kernel(
        _sc_body,
        out_type=jax.ShapeDtypeStruct((B, 16), jnp.float32),
        mesh=mesh,
        scratch_types=[
            pltpu.VMEM((P2,), jnp.float32),
            pltpu.VMEM((P2,), jnp.int32),
            pltpu.VMEM((P2,), jnp.int32),
            pltpu.VMEM((P2,), jnp.int32),
            pltpu.VMEM((P2,), jnp.int32),
            pltpu.VMEM((_NBUCKET,), jnp.int32),
            pltpu.VMEM((_NBUCKET,), jnp.int32),
            pltpu.VMEM((P2,), jnp.float32),
            pltpu.VMEM((16,), jnp.float32),
            pltpu.VMEM((B,), jnp.float32),
        ],
        compiler_params=_SC_PARAMS,
    )
    return f(keys, ce2, npos)


def kernel(pred_conf, pred_loc, priory_boxes, truth):
    pconf_t = jnp.transpose(pred_conf, (0, 2, 1))
    ploc_t = jnp.transpose(pred_loc, (0, 2, 1))
    priors_t = priory_boxes.T

    conf, npos, lloc = _run_match(priors_t, truth, ploc_t)
    keys, ce2, cepos = _run_keys(pconf_t, conf)
    tpart = _run_sc_sort_select(keys, ce2, jnp.reshape(npos, (B,)))

    n = jnp.sum(npos)
    loss_loc = lloc[0, 0] / n
    loss_c = (cepos[0, 0] + jnp.sum(tpart)) / n
    return (loss_loc, loss_c)
